# baseline (device time: 18093 ns/iter reference)
import jax
import jax.numpy as jnp
from jax import lax
from jax.experimental import pallas as pl
from jax.experimental.pallas import tpu as pltpu

N_DEV = 4


def _gelu(y):
    c = 0.7978845608028654
    return 0.5 * y * (1.0 + jnp.tanh(c * (y + 0.044715 * y * y * y)))


def kernel(x, w_mat):
    m, _ = x.shape
    k, n = w_mat.shape
    mc = m // N_DEV
    nh = n // 2

    def body(x_hbm, w_hbm, out_hbm,
             x_ref, w_ref, ostage,
             sbufA, sbufB, rbufA, rbufB, s2A, s2B, r2A, r2B,
             in_sems, out_sems,
             ssemA, ssemB, rsemA, rsemB,
             ssem2A, ssem2B, rsem2A, rsem2B):
        my = lax.axis_index("i")
        pA = jnp.bitwise_xor(my, 1)
        pB = 3 - my

        cp_x = pltpu.make_async_copy(x_hbm, x_ref, in_sems.at[0])
        cp_x.start()
        cp_w = pltpu.make_async_copy(w_hbm, w_ref, in_sems.at[1])
        cp_w.start()

        barrier = pltpu.get_barrier_semaphore()
        for nbr in (pA, pB):
            pl.semaphore_signal(
                barrier, inc=1,
                device_id=(nbr,), device_id_type=pl.DeviceIdType.MESH,
            )
        pl.semaphore_wait(barrier, 2)
        cp_x.wait()
        cp_w.wait()

        def pchunk(c, lo, width):
            return jnp.dot(
                x_ref[pl.ds(c * mc, mc), :],
                w_ref[:, lo:lo + width],
                preferred_element_type=jnp.float32,
            )

        def xfer(src, dst, ssem, rsem, peer):
            return pltpu.make_async_remote_copy(
                src_ref=src, dst_ref=dst, send_sem=ssem, recv_sem=rsem,
                device_id=(peer,), device_id_type=pl.DeviceIdType.MESH,
            )

        sbufA[0] = pchunk(3 - pA, 0, nh).astype(jnp.bfloat16)
        ra0 = xfer(sbufA.at[0], rbufA.at[0], ssemA.at[0], rsemA.at[0], pA)
        ra0.start()
        sbufB[0] = pchunk(jnp.bitwise_xor(pB, 1), nh, nh).astype(jnp.bfloat16)
        rb0 = xfer(sbufB.at[0], rbufB.at[0], ssemB.at[0], rsemB.at[0], pB)
        rb0.start()
        sbufA[1] = pchunk(pA, 0, nh).astype(jnp.bfloat16)
        ra1 = xfer(sbufA.at[1], rbufA.at[1], ssemA.at[1], rsemA.at[1], pA)
        ra1.start()
        sbufB[1] = pchunk(pB, nh, nh).astype(jnp.bfloat16)
        rb1 = xfer(sbufB.at[1], rbufB.at[1], ssemB.at[1], rsemB.at[1], pB)
        rb1.start()

        p_fwd_a = pchunk(3 - my, 0, nh)
        p_fwd_b = pchunk(pA, nh, nh)

        ra0.wait_recv()
        s2A[...] = (rbufA[0].astype(jnp.float32) + p_fwd_a).astype(jnp.bfloat16)
        r2a = xfer(s2A, r2A, ssem2A, rsem2A, pB)
        r2a.start()
        rb0.wait_recv()
        s2B[...] = (rbufB[0].astype(jnp.float32) + p_fwd_b).astype(jnp.bfloat16)
        r2b = xfer(s2B, r2B, ssem2B, rsem2B, pA)
        r2b.start()

        p_own = pchunk(my, 0, n)

        ra1.wait_recv()
        kept_a = rbufA[1].astype(jnp.float32) + p_own[:, :nh]
        rb1.wait_recv()
        kept_b = rbufB[1].astype(jnp.float32) + p_own[:, nh:]

        r2a.wait_recv()
        ostage[:, :nh] = _gelu(kept_a + r2A[...].astype(jnp.float32))
        cp_oa = pltpu.make_async_copy(
            ostage.at[:, 0:nh], out_hbm.at[:, 0:nh], out_sems.at[0]
        )
        cp_oa.start()
        r2b.wait_recv()
        ostage[:, nh:] = _gelu(kept_b + r2B[...].astype(jnp.float32))
        cp_ob = pltpu.make_async_copy(
            ostage.at[:, nh:n], out_hbm.at[:, nh:n], out_sems.at[1]
        )
        cp_ob.start()

        cp_oa.wait()
        cp_ob.wait()
        for r in (ra0, rb0, ra1, rb1, r2a, r2b):
            r.wait_send()

    return pl.pallas_call(
        body,
        out_shape=jax.ShapeDtypeStruct((mc, n), jnp.float32),
        in_specs=[
            pl.BlockSpec(memory_space=pl.ANY),
            pl.BlockSpec(memory_space=pl.ANY),
        ],
        out_specs=pl.BlockSpec(memory_space=pl.ANY),
        scratch_shapes=(
            [
                pltpu.VMEM((m, k), jnp.float32),
                pltpu.VMEM((k, n), jnp.float32),
                pltpu.VMEM((mc, n), jnp.float32),
            ]
            + [pltpu.VMEM((2, mc, nh), jnp.bfloat16)] * 4
            + [pltpu.VMEM((mc, nh), jnp.bfloat16)] * 4
            + [pltpu.SemaphoreType.DMA((2,))] * 2
            + [pltpu.SemaphoreType.DMA((2,))] * 4
            + [pltpu.SemaphoreType.DMA] * 4
        ),
        compiler_params=pltpu.CompilerParams(collective_id=0),
    )(x, w_mat)


# device time: 17942 ns/iter; 1.0084x vs baseline; 1.0084x over previous
import jax
import jax.numpy as jnp
from jax import lax
from jax.experimental import pallas as pl
from jax.experimental.pallas import tpu as pltpu

N_DEV = 4


def _gelu(y):
    c = 0.7978845608028654
    return 0.5 * y * (1.0 + jnp.tanh(c * (y + 0.044715 * y * y * y)))


def kernel(x, w_mat):
    m, _ = x.shape
    _, n = w_mat.shape
    mc = m // N_DEV
    nh = n // 2

    def body(x_ref, w_ref, out_hbm,
             sbufA, sbufB, rbufA, rbufB, s2A, s2B, r2A, r2B,
             ostage, out_sems,
             ssemA, ssemB, rsemA, rsemB,
             ssem2A, ssem2B, rsem2A, rsem2B):
        my = lax.axis_index("i")
        pA = jnp.bitwise_xor(my, 1)
        pB = 3 - my

        barrier = pltpu.get_barrier_semaphore()
        for nbr in (pA, pB):
            pl.semaphore_signal(
                barrier, inc=1,
                device_id=(nbr,), device_id_type=pl.DeviceIdType.MESH,
            )
        pl.semaphore_wait(barrier, 2)

        def pchunk(c, lo, width):
            return jnp.dot(
                x_ref[pl.ds(c * mc, mc), :],
                w_ref[:, lo:lo + width],
                preferred_element_type=jnp.float32,
            )

        def xfer(src, dst, ssem, rsem, peer):
            return pltpu.make_async_remote_copy(
                src_ref=src, dst_ref=dst, send_sem=ssem, recv_sem=rsem,
                device_id=(peer,), device_id_type=pl.DeviceIdType.MESH,
            )

        sbufA[0] = pchunk(3 - pA, 0, nh).astype(jnp.bfloat16)
        ra0 = xfer(sbufA.at[0], rbufA.at[0], ssemA.at[0], rsemA.at[0], pA)
        ra0.start()
        sbufB[0] = pchunk(jnp.bitwise_xor(pB, 1), nh, nh).astype(jnp.bfloat16)
        rb0 = xfer(sbufB.at[0], rbufB.at[0], ssemB.at[0], rsemB.at[0], pB)
        rb0.start()
        sbufA[1] = pchunk(pA, 0, nh).astype(jnp.bfloat16)
        ra1 = xfer(sbufA.at[1], rbufA.at[1], ssemA.at[1], rsemA.at[1], pA)
        ra1.start()
        sbufB[1] = pchunk(pB, nh, nh).astype(jnp.bfloat16)
        rb1 = xfer(sbufB.at[1], rbufB.at[1], ssemB.at[1], rsemB.at[1], pB)
        rb1.start()

        p_fwd_a = pchunk(3 - my, 0, nh)
        p_fwd_b = pchunk(pA, nh, nh)

        ra0.wait_recv()
        s2A[...] = (rbufA[0].astype(jnp.float32) + p_fwd_a).astype(jnp.bfloat16)
        r2a = xfer(s2A, r2A, ssem2A, rsem2A, pB)
        r2a.start()
        rb0.wait_recv()
        s2B[...] = (rbufB[0].astype(jnp.float32) + p_fwd_b).astype(jnp.bfloat16)
        r2b = xfer(s2B, r2B, ssem2B, rsem2B, pA)
        r2b.start()

        p_own = pchunk(my, 0, n)

        ra1.wait_recv()
        kept_a = rbufA[1].astype(jnp.float32) + p_own[:, :nh]
        rb1.wait_recv()
        kept_b = rbufB[1].astype(jnp.float32) + p_own[:, nh:]

        r2a.wait_recv()
        ostage[:, :nh] = _gelu(kept_a + r2A[...].astype(jnp.float32))
        cp_oa = pltpu.make_async_copy(
            ostage.at[:, 0:nh], out_hbm.at[:, 0:nh], out_sems.at[0]
        )
        cp_oa.start()
        r2b.wait_recv()
        ostage[:, nh:] = _gelu(kept_b + r2B[...].astype(jnp.float32))
        cp_ob = pltpu.make_async_copy(
            ostage.at[:, nh:n], out_hbm.at[:, nh:n], out_sems.at[1]
        )
        cp_ob.start()

        cp_oa.wait()
        cp_ob.wait()
        for r in (ra0, rb0, ra1, rb1, r2a, r2b):
            r.wait_send()

    return pl.pallas_call(
        body,
        out_shape=jax.ShapeDtypeStruct((mc, n), jnp.float32),
        in_specs=[
            pl.BlockSpec(memory_space=pltpu.VMEM),
            pl.BlockSpec(memory_space=pltpu.VMEM),
        ],
        out_specs=pl.BlockSpec(memory_space=pl.ANY),
        scratch_shapes=(
            [pltpu.VMEM((2, mc, nh), jnp.bfloat16)] * 4
            + [pltpu.VMEM((mc, nh), jnp.bfloat16)] * 4
            + [
                pltpu.VMEM((mc, n), jnp.float32),
                pltpu.SemaphoreType.DMA((2,)),
            ]
            + [pltpu.SemaphoreType.DMA((2,))] * 4
            + [pltpu.SemaphoreType.DMA] * 4
        ),
        compiler_params=pltpu.CompilerParams(collective_id=0),
    )(x, w_mat)


# device time: 17739 ns/iter; 1.0200x vs baseline; 1.0114x over previous
import jax
import jax.numpy as jnp
from jax import lax
from jax.experimental import pallas as pl
from jax.experimental.pallas import tpu as pltpu

N_DEV = 4


def _gelu(y):
    c = 0.7978845608028654
    return 0.5 * y * (1.0 + jnp.tanh(c * (y + 0.044715 * y * y * y)))


def kernel(x, w_mat):
    m, _ = x.shape
    _, n = w_mat.shape
    mc = m // N_DEV
    nh = n // 2

    def body(x_ref, w_ref, out_hbm,
             sbufA, sbufB, rbufA, rbufB, s2A, s2B, r2A, r2B,
             ostage, out_sems,
             ssemA, ssemB, rsemA, rsemB,
             ssem2A, ssem2B, rsem2A, rsem2B):
        my = lax.axis_index("i")
        pA = jnp.bitwise_xor(my, 1)
        pB = 3 - my

        barrier = pltpu.get_barrier_semaphore()
        for nbr in (pA, pB):
            pl.semaphore_signal(
                barrier, inc=1,
                device_id=(nbr,), device_id_type=pl.DeviceIdType.MESH,
            )
        pl.semaphore_wait(barrier, 2)

        def pchunk(c, lo, width):
            return jnp.dot(
                x_ref[pl.ds(c * mc, mc), :],
                w_ref[:, lo:lo + width],
                preferred_element_type=jnp.float32,
            )

        def xfer(src, dst, ssem, rsem, peer):
            return pltpu.make_async_remote_copy(
                src_ref=src, dst_ref=dst, send_sem=ssem, recv_sem=rsem,
                device_id=(peer,), device_id_type=pl.DeviceIdType.MESH,
            )

        sbufA[0] = pchunk(3 - pA, 0, nh).astype(jnp.bfloat16)
        ra0 = xfer(sbufA.at[0], rbufA.at[0], ssemA.at[0], rsemA.at[0], pA)
        ra0.start()
        sbufB[0] = pchunk(jnp.bitwise_xor(pB, 1), nh, nh).astype(jnp.bfloat16)
        rb0 = xfer(sbufB.at[0], rbufB.at[0], ssemB.at[0], rsemB.at[0], pB)
        rb0.start()
        sbufA[1] = pchunk(pA, 0, nh).astype(jnp.bfloat16)
        ra1 = xfer(sbufA.at[1], rbufA.at[1], ssemA.at[1], rsemA.at[1], pA)
        ra1.start()
        sbufB[1] = pchunk(pB, nh, nh).astype(jnp.bfloat16)
        rb1 = xfer(sbufB.at[1], rbufB.at[1], ssemB.at[1], rsemB.at[1], pB)
        rb1.start()

        p_fwd_a = pchunk(3 - my, 0, nh)
        p_fwd_b = pchunk(pA, nh, nh)

        nq = nh // 2
        ra0.wait_recv()
        acc2a = rbufA[0].astype(jnp.float32) + p_fwd_a
        s2A[0] = acc2a[:, :nq].astype(jnp.bfloat16)
        r2a0 = xfer(s2A.at[0], r2A.at[0], ssem2A.at[0], rsem2A.at[0], pB)
        r2a0.start()
        rb0.wait_recv()
        acc2b = rbufB[0].astype(jnp.float32) + p_fwd_b
        s2B[0] = acc2b[:, :nq].astype(jnp.bfloat16)
        r2b0 = xfer(s2B.at[0], r2B.at[0], ssem2B.at[0], rsem2B.at[0], pA)
        r2b0.start()
        s2A[1] = acc2a[:, nq:].astype(jnp.bfloat16)
        r2a1 = xfer(s2A.at[1], r2A.at[1], ssem2A.at[1], rsem2A.at[1], pB)
        r2a1.start()
        s2B[1] = acc2b[:, nq:].astype(jnp.bfloat16)
        r2b1 = xfer(s2B.at[1], r2B.at[1], ssem2B.at[1], rsem2B.at[1], pA)
        r2b1.start()

        p_own = pchunk(my, 0, n)

        ra1.wait_recv()
        kept_a = rbufA[1].astype(jnp.float32) + p_own[:, :nh]
        rb1.wait_recv()
        kept_b = rbufB[1].astype(jnp.float32) + p_own[:, nh:]

        cps = []
        finals = (
            (r2a0, r2A, 0, kept_a, 0),
            (r2b0, r2B, 0, kept_b, nh),
            (r2a1, r2A, 1, kept_a, nq),
            (r2b1, r2B, 1, kept_b, nh + nq),
        )
        for i, (rd, rbuf, slot, kept, lo) in enumerate(finals):
            klo = lo - nh if lo >= nh else lo
            rd.wait_recv()
            ostage[:, lo:lo + nq] = _gelu(
                kept[:, klo:klo + nq] + rbuf[slot].astype(jnp.float32)
            )
            cp = pltpu.make_async_copy(
                ostage.at[:, lo:lo + nq], out_hbm.at[:, lo:lo + nq],
                out_sems.at[i],
            )
            cp.start()
            cps.append(cp)

        for cp in cps:
            cp.wait()
        for r in (ra0, rb0, ra1, rb1, r2a0, r2a1, r2b0, r2b1):
            r.wait_send()

    return pl.pallas_call(
        body,
        out_shape=jax.ShapeDtypeStruct((mc, n), jnp.float32),
        in_specs=[
            pl.BlockSpec(memory_space=pltpu.VMEM),
            pl.BlockSpec(memory_space=pltpu.VMEM),
        ],
        out_specs=pl.BlockSpec(memory_space=pl.ANY),
        scratch_shapes=(
            [pltpu.VMEM((2, mc, nh), jnp.bfloat16)] * 4
            + [pltpu.VMEM((2, mc, nh // 2), jnp.bfloat16)] * 4
            + [
                pltpu.VMEM((mc, n), jnp.float32),
                pltpu.SemaphoreType.DMA((4,)),
            ]
            + [pltpu.SemaphoreType.DMA((2,))] * 4
            + [pltpu.SemaphoreType.DMA((2,))] * 4
        ),
        compiler_params=pltpu.CompilerParams(collective_id=0),
    )(x, w_mat)
